# Initial kernel scaffold; baseline (speedup 1.0000x reference)
import jax
import jax.numpy as jnp
from jax.experimental import pallas as pl


def kernel(x, edge_index, W1, b1, W2, b2):
    raise NotImplementedError("write your pallas kernel here")


# speedup vs baseline: 37.8506x; 37.8506x over previous
import functools

import jax
import jax.numpy as jnp
from jax import lax
from jax.experimental import pallas as pl
from jax.experimental.pallas import tpu as pltpu
from jax.experimental.pallas import tpu_sc as plsc

N = 10000
E = 320000
D = 128
H = 16
C = 7

NC = 2
NS = 16
NW = NC * NS
CH = 128
NCH = 80
EP = NW * NCH * CH
NPAD = 10240
RPT = NPAD // NS

_mesh = plsc.VectorSubcoreMesh(core_axis_name="c", subcore_axis_name="s",
                               num_cores=NC, num_subcores=NS)
_sc_params = pltpu.CompilerParams(use_tc_tiling_on_sc=False)


@functools.partial(
    pl.kernel,
    out_type=jax.ShapeDtypeStruct((NC, NPAD), jnp.float32),
    mesh=_mesh,
    compiler_params=_sc_params,
    scratch_types=[
        pltpu.VMEM((NCH, CH), jnp.int32),
        pltpu.VMEM((CH,), jnp.float32),
        pltpu.VMEM_SHARED((NPAD,), jnp.float32),
    ],
)
def _sc_degree(dst_hbm, ones_hbm, zeros_hbm, out_hbm, idx_v, ones_v, acc):
    c = lax.axis_index("c")
    s = lax.axis_index("s")
    w = s * NC + c
    pltpu.sync_copy(dst_hbm.at[pl.ds(w * NCH, NCH)], idx_v)
    pltpu.sync_copy(ones_hbm, ones_v)
    pltpu.sync_copy(zeros_hbm.at[pl.ds(s * RPT, RPT)], acc.at[pl.ds(s * RPT, RPT)])
    plsc.subcore_barrier()

    def body(j, carry):
        pltpu.sync_copy(ones_v, acc.at[idx_v.at[j]], add=True)
        return carry

    lax.fori_loop(0, NCH, body, jnp.int32(0))
    plsc.subcore_barrier()
    pltpu.sync_copy(acc.at[pl.ds(s * RPT, RPT)], out_hbm.at[c, pl.ds(s * RPT, RPT)])


@functools.partial(
    pl.kernel,
    out_type=jax.ShapeDtypeStruct((NC, NPAD, H), jnp.float32),
    mesh=_mesh,
    compiler_params=_sc_params,
    scratch_types=[
        pltpu.VMEM((NCH, CH), jnp.int32),
        pltpu.VMEM((NCH, CH), jnp.int32),
        pltpu.VMEM((CH, H), jnp.float32),
        pltpu.VMEM_SHARED((NPAD, H), jnp.float32),
    ],
)
def _sc_row_scatter(table_hbm, src_hbm, dst_hbm, zeros_hbm, out_hbm,
                    src_v, dst_v, rows_v, acc):
    c = lax.axis_index("c")
    s = lax.axis_index("s")
    w = s * NC + c
    pltpu.sync_copy(src_hbm.at[pl.ds(w * NCH, NCH)], src_v)
    pltpu.sync_copy(dst_hbm.at[pl.ds(w * NCH, NCH)], dst_v)
    pltpu.sync_copy(zeros_hbm.at[pl.ds(s * RPT, RPT)], acc.at[pl.ds(s * RPT, RPT)])
    plsc.subcore_barrier()

    def body(j, carry):
        pltpu.sync_copy(table_hbm.at[src_v.at[j]], rows_v)
        pltpu.sync_copy(rows_v, acc.at[dst_v.at[j]], add=True)
        return carry

    lax.fori_loop(0, NCH, body, jnp.int32(0))
    plsc.subcore_barrier()
    pltpu.sync_copy(acc.at[pl.ds(s * RPT, RPT)], out_hbm.at[c, pl.ds(s * RPT, RPT)])


def _tc1_body(x_ref, w1_ref, degp_ref, s1_ref, dinv_ref):
    deg = degp_ref[0, :N] + degp_ref[1, :N] + 1.0
    dinv = lax.rsqrt(deg)[:, None]
    xw = jnp.dot(x_ref[...], w1_ref[...], preferred_element_type=jnp.float32)
    s1_ref[...] = xw * dinv
    dinv_ref[...] = dinv


def _tc2_body(agg_ref, s1_ref, dinv_ref, b1_ref, s2_ref):
    dinv = dinv_ref[...]
    agg = agg_ref[0, :N, :] + agg_ref[1, :N, :] + s1_ref[...]
    h = jnp.maximum(agg * dinv + b1_ref[...][None, :], 0.0)
    s2_ref[...] = h * dinv


def _tc3_body(agg_ref, s2_ref, dinv_ref, w2_ref, b2_ref, out_ref):
    agg = agg_ref[0, :N, :] + agg_ref[1, :N, :] + s2_ref[...]
    pre = agg * dinv_ref[...]
    o = jnp.dot(pre, w2_ref[...], preferred_element_type=jnp.float32)
    o = o + b2_ref[...][None, :]
    m = jnp.max(o, axis=1, keepdims=True)
    lse = jnp.log(jnp.sum(jnp.exp(o - m), axis=1, keepdims=True)) + m
    out_ref[...] = o - lse


def kernel(x, edge_index, W1, b1, W2, b2):
    src = edge_index[0]
    dst = edge_index[1]
    npadr = jnp.arange(EP - E, dtype=jnp.int32)
    src_p = jnp.concatenate([src, npadr % N]).reshape(NW * NCH, CH)
    dst_p = jnp.concatenate([dst, N + npadr % (NPAD - N)]).reshape(NW * NCH, CH)

    ones_ch = jnp.ones((CH,), jnp.float32)
    zeros1 = jnp.zeros((NPAD,), jnp.float32)
    zeros2 = jnp.zeros((NPAD, H), jnp.float32)

    degp = _sc_degree(dst_p, ones_ch, zeros1)

    s1, dinv = pl.pallas_call(
        _tc1_body,
        out_shape=(jax.ShapeDtypeStruct((N, H), jnp.float32),
                   jax.ShapeDtypeStruct((N, 1), jnp.float32)),
    )(x, W1, degp)

    agg1 = _sc_row_scatter(s1, src_p, dst_p, zeros2)

    s2 = pl.pallas_call(
        _tc2_body,
        out_shape=jax.ShapeDtypeStruct((N, H), jnp.float32),
    )(agg1, s1, dinv, b1)

    agg2 = _sc_row_scatter(s2, src_p, dst_p, zeros2)

    out = pl.pallas_call(
        _tc3_body,
        out_shape=jax.ShapeDtypeStruct((N, C), jnp.float32),
    )(agg2, s2, dinv, W2, b2)
    return out
